# baseline (speedup 1.0000x reference)
import functools

import jax
import jax.numpy as jnp
from jax import lax
from jax.experimental import pallas as pl
from jax.experimental.pallas import tpu as pltpu
from jax.experimental.pallas import tpu_sc as plsc

N = 10000
IN_F = 128
HID = 128
NC = 40
E = 160000

NCORE = 2
NSUB = 16
K = 128
EPT = 10240
EPAD = EPT * NSUB
NBLK = EPT // K
ACC_ROWS = 10240
RPT = ACC_ROWS // NSUB
ZROWS = 64

W0 = 144
W1 = 48

RB = 1000
G = N // RB


NBUF = 4


@functools.lru_cache(maxsize=None)
def _make_sc_agg(width, n_scale_chunks, k):
  nblk = EPT // k
  mesh = plsc.VectorSubcoreMesh(core_axis_name="c", subcore_axis_name="s",
                                num_cores=NCORE, num_subcores=NSUB)

  @functools.partial(
      pl.kernel,
      out_type=(
          jax.ShapeDtypeStruct((ACC_ROWS, width), jnp.float32),
          jax.ShapeDtypeStruct((ACC_ROWS, width), jnp.float32),
      ),
      mesh=mesh,
      scratch_types=[
          pltpu.VMEM_SHARED((ACC_ROWS, width), jnp.float32),
          [pltpu.VMEM((3, k), jnp.int32)] * NBUF,
          [pltpu.VMEM((k, width), jnp.float32)] * NBUF,
          [pltpu.SemaphoreType.DMA] * NBUF,
          [pltpu.SemaphoreType.DMA] * NBUF,
          [pltpu.SemaphoreType.DMA] * NBUF,
      ],
      compiler_params=pltpu.CompilerParams(use_tc_tiling_on_sc=False,
                                           needs_layout_passes=False),
  )
  def sc_agg(t0, t1, ec0, ec1, zeros_h, out0, out1,
             acc_sh, ecb, rows, isem, gsem, ssem):
    c = lax.axis_index("c")
    s = lax.axis_index("s")

    def edge_loop(ec_h, t_h):
      bb = s * nblk

      def i_desc(b, p):
        return pltpu.make_async_copy(ec_h.at[bb + b], ecb[p], isem[p])

      def g_desc(b, p):
        return pltpu.make_async_copy(t_h.at[ecb[p].at[0]], rows[p], gsem[p])

      def s_start(b, p):
        pltpu.async_copy(rows[p], acc_sh.at[ecb[p].at[1]], ssem[p], add=True)

      def s_wait(b, p):
        pltpu.make_async_copy(rows[p], acc_sh.at[ecb[p].at[1]],
                              ssem[p]).wait()

      def scale(b, p):
        @plsc.parallel_loop(0, k // 16)
        def _(g):
          wv = plsc.bitcast(ecb[p][2, pl.ds(g * 16, 16)], jnp.float32)
          for lane in range(16):
            wbc = wv.at[jnp.full((16,), lane, jnp.int32)].get(
                mode="promise_in_bounds")
            e = g * 16 + lane
            for j in range(n_scale_chunks):
              sl = pl.ds(j * 16, 16)
              rows[p][e, sl] = rows[p][e, sl] * wbc

      for p in range(NBUF):
        i_desc(p, p).start()
      i_desc(0, 0).wait()
      g_desc(0, 0).start()
      pltpu.sync_copy(zeros_h.at[pl.ds(s * RPT, RPT)],
                      acc_sh.at[pl.ds(s * RPT, RPT)])
      plsc.subcore_barrier()

      def quad(i, _):
        for u in range(NBUF):
          b = i * NBUF + u
          un = (u + 1) % NBUF
          uq = (u + NBUF - 1) % NBUF

          @pl.when(b + 1 < nblk)
          def _():
            i_desc(b + 1, un).wait()
            g_desc(b + 1, un).start()

          g_desc(b, u).wait()
          scale(b, u)
          s_start(b, u)

          @pl.when(jnp.logical_and(b >= 1, b + NBUF - 1 < nblk))
          def _():
            s_wait(b - 1, uq)
            i_desc(b + NBUF - 1, uq).start()
        return 0
      lax.fori_loop(0, nblk // NBUF, quad, 0)

      for j in range(NBUF):
        bj = nblk - NBUF + j
        s_wait(bj, bj % NBUF)

    @pl.when(c == 0)
    def _():
      edge_loop(ec0, t0)

    @pl.when(c == 1)
    def _():
      edge_loop(ec1, t1)

    plsc.subcore_barrier()

    def out_copy(out_h):
      pltpu.sync_copy(acc_sh.at[pl.ds(s * RPT, RPT)],
                      out_h.at[pl.ds(s * RPT, RPT)])

    @pl.when(c == 0)
    def _():
      out_copy(out0)

    @pl.when(c == 1)
    def _():
      out_copy(out1)

  return sc_agg


def _tc1_body(x_ref, w_ref, b_ref, t0_ref, t1_ref, xs_ref):
  y = jnp.dot(x_ref[...], w_ref[...], preferred_element_type=jnp.float32,
              precision=jax.lax.Precision.HIGHEST)
  ones = jnp.ones((RB, 1), jnp.float32)
  zpad = jnp.zeros((RB, W0 - HID - 1), jnp.float32)
  t0_ref[...] = jnp.concatenate([y[:, :HID], ones, zpad], axis=1)
  t1_ref[...] = jnp.concatenate([y[:, HID:2 * HID], ones, zpad], axis=1)
  xs_ref[...] = y[:, 2 * HID:] + b_ref[...]


def _tc2_body(xs_ref, a0_ref, a1_ref, w_ref, b_ref, t0_ref, t1_ref, hs_ref):
  cnt0 = a0_ref[:, HID:HID + 1]
  cnt1 = a1_ref[:, HID:HID + 1]
  inv0 = 1.0 / jnp.maximum(cnt0, 1.0)
  inv1 = 1.0 / jnp.maximum(cnt1, 1.0)
  h = jax.nn.relu(xs_ref[...] + a0_ref[:, :HID] * inv0 + a1_ref[:, :HID] * inv1)
  y = jnp.dot(h, w_ref[...], preferred_element_type=jnp.float32,
              precision=jax.lax.Precision.HIGHEST)
  zpad = jnp.zeros((RB, W1 - NC), jnp.float32)
  t0_ref[...] = jnp.concatenate([y[:, :NC], zpad], axis=1)
  t1_ref[...] = jnp.concatenate([y[:, NC:2 * NC], zpad], axis=1)
  zpad2 = jnp.zeros((RB, W1 - NC - 2), jnp.float32)
  hs_ref[...] = jnp.concatenate(
      [y[:, 2 * NC:3 * NC] + b_ref[...], inv0, inv1, zpad2], axis=1)


def _tc3_body(hs_ref, a0_ref, a1_ref, out_ref):
  inv0 = hs_ref[:, NC:NC + 1]
  inv1 = hs_ref[:, NC + 1:NC + 2]
  out_ref[...] = (hs_ref[:, :NC]
                  + a0_ref[:, :NC] * inv0
                  + a1_ref[:, :NC] * inv1)


def _row_spec(w):
  return pl.BlockSpec((RB, w), lambda i: (i, 0))


def _full_spec(r, w):
  return pl.BlockSpec((r, w), lambda i: (0, 0))


_tc1 = pl.pallas_call(
    _tc1_body,
    grid=(G,),
    in_specs=[_row_spec(IN_F), _full_spec(IN_F, 2 * HID + IN_F), _full_spec(1, HID)],
    out_specs=[_row_spec(W0), _row_spec(W0), _row_spec(HID)],
    out_shape=[
        jax.ShapeDtypeStruct((N, W0), jnp.float32),
        jax.ShapeDtypeStruct((N, W0), jnp.float32),
        jax.ShapeDtypeStruct((N, HID), jnp.float32),
    ],
)

_tc2 = pl.pallas_call(
    _tc2_body,
    grid=(G,),
    in_specs=[_row_spec(HID), _row_spec(W0), _row_spec(W0),
              _full_spec(HID, 128), _full_spec(1, NC)],
    out_specs=[_row_spec(W1), _row_spec(W1), _row_spec(W1)],
    out_shape=[
        jax.ShapeDtypeStruct((N, W1), jnp.float32),
        jax.ShapeDtypeStruct((N, W1), jnp.float32),
        jax.ShapeDtypeStruct((N, W1), jnp.float32),
    ],
)

_tc3 = pl.pallas_call(
    _tc3_body,
    grid=(G,),
    in_specs=[_row_spec(W1), _row_spec(W1), _row_spec(W1)],
    out_specs=pl.BlockSpec((RB, NC), lambda i: (i, 0)),
    out_shape=jax.ShapeDtypeStruct((N, NC), jnp.float32),
)


def _pad_edges(edge_index, edge_weight, k):
  pad = EPAD - E
  src = jnp.concatenate(
      [edge_index[0].astype(jnp.int32), jnp.zeros((pad,), jnp.int32)])
  dst = jnp.concatenate(
      [edge_index[1].astype(jnp.int32), jnp.full((pad,), N, jnp.int32)])
  w = jnp.concatenate([edge_weight, jnp.zeros((pad,), jnp.float32)])
  nblkt = EPAD // k
  return jnp.stack([src.reshape(nblkt, k), dst.reshape(nblkt, k),
                    w.view(jnp.int32).reshape(nblkt, k)], axis=1)


K0 = 64
K1 = 128


def kernel(x, edge_index_r0, edge_index_r1, edge_weight_r0, edge_weight_r1,
           Ws0_r0, Wn0_r0, b0_r0, Ws0_r1, Wn0_r1, b0_r1,
           Ws1_r0, Wn1_r0, b1_r0, Ws1_r1, Wn1_r1, b1_r1):
  ec0_r0 = _pad_edges(edge_index_r0, edge_weight_r0, K0)
  ec0_r1 = _pad_edges(edge_index_r1, edge_weight_r1, K0)
  ec1_r0 = _pad_edges(edge_index_r0, edge_weight_r0, K1)
  ec1_r1 = _pad_edges(edge_index_r1, edge_weight_r1, K1)
  z0 = jnp.zeros((ACC_ROWS, W0), jnp.float32)
  z1 = jnp.zeros((ACC_ROWS, W1), jnp.float32)

  w0all = jnp.concatenate([Wn0_r0, Wn0_r1, Ws0_r0 + Ws0_r1], axis=1)
  b0sum = (b0_r0 + b0_r1)[None, :]
  w1all = jnp.concatenate(
      [Wn1_r0, Wn1_r1, Ws1_r0 + Ws1_r1,
       jnp.zeros((HID, 128 - 3 * NC), jnp.float32)], axis=1)
  b1sum = (b1_r0 + b1_r1)[None, :]

  t0_r0, t0_r1, xs = _tc1(x, w0all, b0sum)
  acc0_r0, acc0_r1 = _make_sc_agg(W0, 8, K0)(t0_r0, t0_r1, ec0_r0, ec0_r1, z0)
  t1_r0, t1_r1, hs = _tc2(xs, acc0_r0, acc0_r1, w1all, b1sum)
  acc1_r0, acc1_r1 = _make_sc_agg(W1, 3, K1)(t1_r0, t1_r1, ec1_r0, ec1_r1, z1)
  return _tc3(hs, acc1_r0, acc1_r1)


# speedup vs baseline: 1.0068x; 1.0068x over previous
import functools

import jax
import jax.numpy as jnp
from jax import lax
from jax.experimental import pallas as pl
from jax.experimental.pallas import tpu as pltpu
from jax.experimental.pallas import tpu_sc as plsc

N = 10000
IN_F = 128
HID = 128
NC = 40
E = 160000

NCORE = 2
NSUB = 16
K = 128
EPT = 10240
EPAD = EPT * NSUB
NBLK = EPT // K
ACC_ROWS = 10240
RPT = ACC_ROWS // NSUB
ZROWS = 64

W0 = 144
W1 = 48

RB = 1000
G = N // RB


NBUF = 4


@functools.lru_cache(maxsize=None)
def _make_sc_agg(width, n_scale_chunks, k, bulk_ec=False):
  nblk = EPT // k
  mesh = plsc.VectorSubcoreMesh(core_axis_name="c", subcore_axis_name="s",
                                num_cores=NCORE, num_subcores=NSUB)

  @functools.partial(
      pl.kernel,
      out_type=(
          jax.ShapeDtypeStruct((ACC_ROWS, width), jnp.float32),
          jax.ShapeDtypeStruct((ACC_ROWS, width), jnp.float32),
      ),
      mesh=mesh,
      scratch_types=[
          pltpu.VMEM_SHARED((ACC_ROWS, width), jnp.float32),
          (pltpu.VMEM((EPT // k, 3, k), jnp.int32) if bulk_ec
           else [pltpu.VMEM((3, k), jnp.int32)] * NBUF),
          [pltpu.VMEM((k, width), jnp.float32)] * NBUF,
          [pltpu.SemaphoreType.DMA] * NBUF,
          [pltpu.SemaphoreType.DMA] * NBUF,
          [pltpu.SemaphoreType.DMA] * NBUF,
      ],
      compiler_params=pltpu.CompilerParams(use_tc_tiling_on_sc=False,
                                           needs_layout_passes=False),
  )
  def sc_agg(t0, t1, ec0, ec1, zeros_h, out0, out1,
             acc_sh, ecb, rows, isem, gsem, ssem):
    c = lax.axis_index("c")
    s = lax.axis_index("s")

    def edge_loop(ec_h, t_h):
      bb = s * nblk

      def src_ref(b, p):
        return ecb.at[b].at[0] if bulk_ec else ecb[p].at[0]

      def dst_ref(b, p):
        return ecb.at[b].at[1] if bulk_ec else ecb[p].at[1]

      def wrow(b, p, sl):
        return ecb[b, 2, sl] if bulk_ec else ecb[p][2, sl]

      def i_desc(b, p):
        return pltpu.make_async_copy(ec_h.at[bb + b], ecb[p], isem[p])

      def g_desc(b, p):
        return pltpu.make_async_copy(t_h.at[src_ref(b, p)], rows[p], gsem[p])

      def s_start(b, p):
        pltpu.async_copy(rows[p], acc_sh.at[dst_ref(b, p)], ssem[p], add=True)

      def s_wait(b, p):
        pltpu.make_async_copy(rows[p], acc_sh.at[dst_ref(b, p)],
                              ssem[p]).wait()

      def scale(b, p):
        @plsc.parallel_loop(0, k // 16)
        def _(g):
          wv = plsc.bitcast(wrow(b, p, pl.ds(g * 16, 16)), jnp.float32)
          for lane in range(16):
            wbc = wv.at[jnp.full((16,), lane, jnp.int32)].get(
                mode="promise_in_bounds")
            e = g * 16 + lane
            for j in range(n_scale_chunks):
              sl = pl.ds(j * 16, 16)
              rows[p][e, sl] = rows[p][e, sl] * wbc

      if bulk_ec:
        pltpu.sync_copy(ec_h.at[pl.ds(bb, nblk)], ecb)
        for p in range(NBUF):
          g_desc(p, p).start()
      else:
        for p in range(NBUF):
          i_desc(p, p).start()
        i_desc(0, 0).wait()
        g_desc(0, 0).start()
      pltpu.sync_copy(zeros_h.at[pl.ds(s * RPT, RPT)],
                      acc_sh.at[pl.ds(s * RPT, RPT)])
      plsc.subcore_barrier()

      def quad(i, _):
        for u in range(NBUF):
          b = i * NBUF + u
          un = (u + 1) % NBUF
          uq = (u + NBUF - 1) % NBUF

          if not bulk_ec:
            @pl.when(b + 1 < nblk)
            def _():
              i_desc(b + 1, un).wait()
              g_desc(b + 1, un).start()

          g_desc(b, u).wait()
          scale(b, u)
          s_start(b, u)

          @pl.when(jnp.logical_and(b >= 1, b + NBUF - 1 < nblk))
          def _():
            s_wait(b - 1, uq)
            if bulk_ec:
              g_desc(b + NBUF - 1, uq).start()
            else:
              i_desc(b + NBUF - 1, uq).start()
        return 0
      lax.fori_loop(0, nblk // NBUF, quad, 0)

      for j in range(NBUF):
        bj = nblk - NBUF + j
        s_wait(bj, bj % NBUF)

    @pl.when(c == 0)
    def _():
      edge_loop(ec0, t0)

    @pl.when(c == 1)
    def _():
      edge_loop(ec1, t1)

    plsc.subcore_barrier()

    def out_copy(out_h):
      pltpu.sync_copy(acc_sh.at[pl.ds(s * RPT, RPT)],
                      out_h.at[pl.ds(s * RPT, RPT)])

    @pl.when(c == 0)
    def _():
      out_copy(out0)

    @pl.when(c == 1)
    def _():
      out_copy(out1)

  return sc_agg


def _tc1_body(x_ref, w_ref, b_ref, t0_ref, t1_ref, xs_ref):
  y = jnp.dot(x_ref[...], w_ref[...], preferred_element_type=jnp.float32,
              precision=jax.lax.Precision.HIGHEST)
  ones = jnp.ones((RB, 1), jnp.float32)
  zpad = jnp.zeros((RB, W0 - HID - 1), jnp.float32)
  t0_ref[...] = jnp.concatenate([y[:, :HID], ones, zpad], axis=1)
  t1_ref[...] = jnp.concatenate([y[:, HID:2 * HID], ones, zpad], axis=1)
  xs_ref[...] = y[:, 2 * HID:] + b_ref[...]


def _tc2_body(xs_ref, a0_ref, a1_ref, w_ref, b_ref, t0_ref, t1_ref, hs_ref):
  cnt0 = a0_ref[:, HID:HID + 1]
  cnt1 = a1_ref[:, HID:HID + 1]
  inv0 = 1.0 / jnp.maximum(cnt0, 1.0)
  inv1 = 1.0 / jnp.maximum(cnt1, 1.0)
  h = jax.nn.relu(xs_ref[...] + a0_ref[:, :HID] * inv0 + a1_ref[:, :HID] * inv1)
  y = jnp.dot(h, w_ref[...], preferred_element_type=jnp.float32,
              precision=jax.lax.Precision.HIGHEST)
  zpad = jnp.zeros((RB, W1 - NC), jnp.float32)
  t0_ref[...] = jnp.concatenate([y[:, :NC], zpad], axis=1)
  t1_ref[...] = jnp.concatenate([y[:, NC:2 * NC], zpad], axis=1)
  zpad2 = jnp.zeros((RB, W1 - NC - 2), jnp.float32)
  hs_ref[...] = jnp.concatenate(
      [y[:, 2 * NC:3 * NC] + b_ref[...], inv0, inv1, zpad2], axis=1)


def _tc3_body(hs_ref, a0_ref, a1_ref, out_ref):
  inv0 = hs_ref[:, NC:NC + 1]
  inv1 = hs_ref[:, NC + 1:NC + 2]
  out_ref[...] = (hs_ref[:, :NC]
                  + a0_ref[:, :NC] * inv0
                  + a1_ref[:, :NC] * inv1)


def _row_spec(w):
  return pl.BlockSpec((RB, w), lambda i: (i, 0))


def _full_spec(r, w):
  return pl.BlockSpec((r, w), lambda i: (0, 0))


_tc1 = pl.pallas_call(
    _tc1_body,
    grid=(G,),
    in_specs=[_row_spec(IN_F), _full_spec(IN_F, 2 * HID + IN_F), _full_spec(1, HID)],
    out_specs=[_row_spec(W0), _row_spec(W0), _row_spec(HID)],
    out_shape=[
        jax.ShapeDtypeStruct((N, W0), jnp.float32),
        jax.ShapeDtypeStruct((N, W0), jnp.float32),
        jax.ShapeDtypeStruct((N, HID), jnp.float32),
    ],
)

_tc2 = pl.pallas_call(
    _tc2_body,
    grid=(G,),
    in_specs=[_row_spec(HID), _row_spec(W0), _row_spec(W0),
              _full_spec(HID, 128), _full_spec(1, NC)],
    out_specs=[_row_spec(W1), _row_spec(W1), _row_spec(W1)],
    out_shape=[
        jax.ShapeDtypeStruct((N, W1), jnp.float32),
        jax.ShapeDtypeStruct((N, W1), jnp.float32),
        jax.ShapeDtypeStruct((N, W1), jnp.float32),
    ],
)

_tc3 = pl.pallas_call(
    _tc3_body,
    grid=(G,),
    in_specs=[_row_spec(W1), _row_spec(W1), _row_spec(W1)],
    out_specs=pl.BlockSpec((RB, NC), lambda i: (i, 0)),
    out_shape=jax.ShapeDtypeStruct((N, NC), jnp.float32),
)


def _pad_edges(edge_index, edge_weight, k):
  pad = EPAD - E
  src = jnp.concatenate(
      [edge_index[0].astype(jnp.int32), jnp.zeros((pad,), jnp.int32)])
  dst = jnp.concatenate(
      [edge_index[1].astype(jnp.int32), jnp.full((pad,), N, jnp.int32)])
  w = jnp.concatenate([edge_weight, jnp.zeros((pad,), jnp.float32)])
  nblkt = EPAD // k
  return jnp.stack([src.reshape(nblkt, k), dst.reshape(nblkt, k),
                    w.view(jnp.int32).reshape(nblkt, k)], axis=1)


K0 = 64
K1 = 128


def kernel(x, edge_index_r0, edge_index_r1, edge_weight_r0, edge_weight_r1,
           Ws0_r0, Wn0_r0, b0_r0, Ws0_r1, Wn0_r1, b0_r1,
           Ws1_r0, Wn1_r0, b1_r0, Ws1_r1, Wn1_r1, b1_r1):
  ec0_r0 = _pad_edges(edge_index_r0, edge_weight_r0, K0)
  ec0_r1 = _pad_edges(edge_index_r1, edge_weight_r1, K0)
  ec1_r0 = _pad_edges(edge_index_r0, edge_weight_r0, K1)
  ec1_r1 = _pad_edges(edge_index_r1, edge_weight_r1, K1)
  z0 = jnp.zeros((ACC_ROWS, W0), jnp.float32)
  z1 = jnp.zeros((ACC_ROWS, W1), jnp.float32)

  w0all = jnp.concatenate([Wn0_r0, Wn0_r1, Ws0_r0 + Ws0_r1], axis=1)
  b0sum = (b0_r0 + b0_r1)[None, :]
  w1all = jnp.concatenate(
      [Wn1_r0, Wn1_r1, Ws1_r0 + Ws1_r1,
       jnp.zeros((HID, 128 - 3 * NC), jnp.float32)], axis=1)
  b1sum = (b1_r0 + b1_r1)[None, :]

  t0_r0, t0_r1, xs = _tc1(x, w0all, b0sum)
  acc0_r0, acc0_r1 = _make_sc_agg(W0, 8, K0)(t0_r0, t0_r1, ec0_r0, ec0_r1, z0)
  t1_r0, t1_r1, hs = _tc2(xs, acc0_r0, acc0_r1, w1all, b1sum)
  acc1_r0, acc1_r1 = _make_sc_agg(W1, 3, K1, True)(t1_r0, t1_r1, ec1_r0, ec1_r1, z1)
  return _tc3(hs, acc1_r0, acc1_r1)


# speedup vs baseline: 1.1202x; 1.1126x over previous
import functools

import jax
import jax.numpy as jnp
from jax import lax
from jax.experimental import pallas as pl
from jax.experimental.pallas import tpu as pltpu
from jax.experimental.pallas import tpu_sc as plsc

N = 10000
IN_F = 128
HID = 128
NC = 40
E = 160000

NCORE = 2
NSUB = 16
K = 128
EPT = 10240
EPAD = EPT * NSUB
NBLK = EPT // K
ACC_ROWS = 10240
RPT = ACC_ROWS // NSUB
ZROWS = 64

W0 = 144
W1 = 48

RB = 1000
G = N // RB


NBUF = 4
NEC = 4


@functools.lru_cache(maxsize=None)
def _make_sc_agg(width, n_scale_chunks, k, bulk_ec=False):
  nblk = EPT // k
  nrow = NBUF if bulk_ec else 2
  mesh = plsc.VectorSubcoreMesh(core_axis_name="c", subcore_axis_name="s",
                                num_cores=NCORE, num_subcores=NSUB)

  @functools.partial(
      pl.kernel,
      out_type=(
          jax.ShapeDtypeStruct((ACC_ROWS, width), jnp.float32),
          jax.ShapeDtypeStruct((ACC_ROWS, width), jnp.float32),
      ),
      mesh=mesh,
      scratch_types=[
          pltpu.VMEM_SHARED((ACC_ROWS, width), jnp.float32),
          (pltpu.VMEM((EPT // k, 3, k), jnp.int32) if bulk_ec
           else [pltpu.VMEM((3, k), jnp.int32)] * NEC),
          [pltpu.VMEM((k, width), jnp.float32)] * nrow,
          [pltpu.SemaphoreType.DMA] * NEC,
          [pltpu.SemaphoreType.DMA] * nrow,
          [pltpu.SemaphoreType.DMA] * nrow,
      ],
      compiler_params=pltpu.CompilerParams(use_tc_tiling_on_sc=False,
                                           needs_layout_passes=False),
  )
  def sc_agg(t0, t1, ec0, ec1, zeros_h, out0, out1,
             acc_sh, ecb, rows, isem, gsem, ssem):
    c = lax.axis_index("c")
    s = lax.axis_index("s")

    def edge_loop(ec_h, t_h):
      bb = s * nblk

      def src_ref(b, ep):
        return ecb.at[b].at[0] if bulk_ec else ecb[ep].at[0]

      def dst_ref(b, ep):
        return ecb.at[b].at[1] if bulk_ec else ecb[ep].at[1]

      def wrow(b, ep, sl):
        return ecb[b, 2, sl] if bulk_ec else ecb[ep][2, sl]

      def i_desc(b, ep):
        return pltpu.make_async_copy(ec_h.at[bb + b], ecb[ep], isem[ep])

      def g_desc(b, p, ep=None):
        ep = p if ep is None else ep
        return pltpu.make_async_copy(t_h.at[src_ref(b, ep)], rows[p], gsem[p])

      def s_start(b, p, ep=None):
        ep = p if ep is None else ep
        pltpu.async_copy(rows[p], acc_sh.at[dst_ref(b, ep)], ssem[p], add=True)

      def s_wait(b, p, ep=None):
        ep = p if ep is None else ep
        pltpu.make_async_copy(rows[p], acc_sh.at[dst_ref(b, ep)],
                              ssem[p]).wait()

      def scale(b, p, ep=None):
        ep = p if ep is None else ep

        @plsc.parallel_loop(0, k // 16)
        def _(g):
          wv = plsc.bitcast(wrow(b, ep, pl.ds(g * 16, 16)), jnp.float32)
          for lane in range(16):
            wbc = wv.at[jnp.full((16,), lane, jnp.int32)].get(
                mode="promise_in_bounds")
            e = g * 16 + lane
            for j in range(n_scale_chunks):
              sl = pl.ds(j * 16, 16)
              rows[p][e, sl] = rows[p][e, sl] * wbc

      if bulk_ec:
        pltpu.sync_copy(ec_h.at[pl.ds(bb, nblk)], ecb)
        for p in range(NBUF):
          g_desc(p, p).start()
      else:
        for p in range(NEC - 1):
          i_desc(p, p).start()
        i_desc(0, 0).wait()
        g_desc(0, 0).start()
      pltpu.sync_copy(zeros_h.at[pl.ds(s * RPT, RPT)],
                      acc_sh.at[pl.ds(s * RPT, RPT)])
      plsc.subcore_barrier()

      if bulk_ec:
        def quad(i, _):
          for u in range(NBUF):
            b = i * NBUF + u
            uq = (u + NBUF - 1) % NBUF
            g_desc(b, u).wait()
            scale(b, u)
            s_start(b, u)

            @pl.when(jnp.logical_and(b >= 1, b + NBUF - 1 < nblk))
            def _():
              s_wait(b - 1, uq)
              g_desc(b + NBUF - 1, uq).start()
          return 0
        lax.fori_loop(0, nblk // NBUF, quad, 0)

        for j in range(NBUF):
          bj = nblk - NBUF + j
          s_wait(bj, bj % NBUF)
      else:
        def quad(i, _):
          for u in range(NEC):
            b = i * NEC + u
            p = u % 2
            q = 1 - p

            @pl.when(b >= 1)
            def _():
              s_wait(b - 1, q, (u + NEC - 1) % NEC)

            @pl.when(b + 1 < nblk)
            def _():
              i_desc(b + 1, (u + 1) % NEC).wait()
              g_desc(b + 1, q, (u + 1) % NEC).start()

            @pl.when(b + NEC - 1 < nblk)
            def _():
              i_desc(b + NEC - 1, (u + NEC - 1) % NEC).start()

            g_desc(b, p, u).wait()
            scale(b, p, u)
            s_start(b, p, u)
          return 0
        lax.fori_loop(0, nblk // NEC, quad, 0)
        s_wait(nblk - 1, (nblk - 1) % 2, (nblk - 1) % NEC)

    @pl.when(c == 0)
    def _():
      edge_loop(ec0, t0)

    @pl.when(c == 1)
    def _():
      edge_loop(ec1, t1)

    plsc.subcore_barrier()

    def out_copy(out_h):
      pltpu.sync_copy(acc_sh.at[pl.ds(s * RPT, RPT)],
                      out_h.at[pl.ds(s * RPT, RPT)])

    @pl.when(c == 0)
    def _():
      out_copy(out0)

    @pl.when(c == 1)
    def _():
      out_copy(out1)

  return sc_agg


def _tc1_body(x_ref, w_ref, b_ref, t0_ref, t1_ref, xs_ref):
  y = jnp.dot(x_ref[...], w_ref[...], preferred_element_type=jnp.float32,
              precision=jax.lax.Precision.HIGHEST)
  ones = jnp.ones((RB, 1), jnp.float32)
  zpad = jnp.zeros((RB, W0 - HID - 1), jnp.float32)
  t0_ref[...] = jnp.concatenate([y[:, :HID], ones, zpad], axis=1)
  t1_ref[...] = jnp.concatenate([y[:, HID:2 * HID], ones, zpad], axis=1)
  xs_ref[...] = y[:, 2 * HID:] + b_ref[...]


def _tc2_body(xs_ref, a0_ref, a1_ref, w_ref, b_ref, t0_ref, t1_ref, hs_ref):
  cnt0 = a0_ref[:, HID:HID + 1]
  cnt1 = a1_ref[:, HID:HID + 1]
  inv0 = 1.0 / jnp.maximum(cnt0, 1.0)
  inv1 = 1.0 / jnp.maximum(cnt1, 1.0)
  h = jax.nn.relu(xs_ref[...] + a0_ref[:, :HID] * inv0 + a1_ref[:, :HID] * inv1)
  y = jnp.dot(h, w_ref[...], preferred_element_type=jnp.float32,
              precision=jax.lax.Precision.HIGHEST)
  zpad = jnp.zeros((RB, W1 - NC), jnp.float32)
  t0_ref[...] = jnp.concatenate([y[:, :NC], zpad], axis=1)
  t1_ref[...] = jnp.concatenate([y[:, NC:2 * NC], zpad], axis=1)
  zpad2 = jnp.zeros((RB, W1 - NC - 2), jnp.float32)
  hs_ref[...] = jnp.concatenate(
      [y[:, 2 * NC:3 * NC] + b_ref[...], inv0, inv1, zpad2], axis=1)


def _tc3_body(hs_ref, a0_ref, a1_ref, out_ref):
  inv0 = hs_ref[:, NC:NC + 1]
  inv1 = hs_ref[:, NC + 1:NC + 2]
  out_ref[...] = (hs_ref[:, :NC]
                  + a0_ref[:, :NC] * inv0
                  + a1_ref[:, :NC] * inv1)


def _row_spec(w):
  return pl.BlockSpec((RB, w), lambda i: (i, 0))


def _full_spec(r, w):
  return pl.BlockSpec((r, w), lambda i: (0, 0))


_tc1 = pl.pallas_call(
    _tc1_body,
    grid=(G,),
    in_specs=[_row_spec(IN_F), _full_spec(IN_F, 2 * HID + IN_F), _full_spec(1, HID)],
    out_specs=[_row_spec(W0), _row_spec(W0), _row_spec(HID)],
    out_shape=[
        jax.ShapeDtypeStruct((N, W0), jnp.float32),
        jax.ShapeDtypeStruct((N, W0), jnp.float32),
        jax.ShapeDtypeStruct((N, HID), jnp.float32),
    ],
)

_tc2 = pl.pallas_call(
    _tc2_body,
    grid=(G,),
    in_specs=[_row_spec(HID), _row_spec(W0), _row_spec(W0),
              _full_spec(HID, 128), _full_spec(1, NC)],
    out_specs=[_row_spec(W1), _row_spec(W1), _row_spec(W1)],
    out_shape=[
        jax.ShapeDtypeStruct((N, W1), jnp.float32),
        jax.ShapeDtypeStruct((N, W1), jnp.float32),
        jax.ShapeDtypeStruct((N, W1), jnp.float32),
    ],
)

_tc3 = pl.pallas_call(
    _tc3_body,
    grid=(G,),
    in_specs=[_row_spec(W1), _row_spec(W1), _row_spec(W1)],
    out_specs=pl.BlockSpec((RB, NC), lambda i: (i, 0)),
    out_shape=jax.ShapeDtypeStruct((N, NC), jnp.float32),
)


def _pad_edges(edge_index, edge_weight, k):
  pad = EPAD - E
  src = jnp.concatenate(
      [edge_index[0].astype(jnp.int32), jnp.zeros((pad,), jnp.int32)])
  dst = jnp.concatenate(
      [edge_index[1].astype(jnp.int32), jnp.full((pad,), N, jnp.int32)])
  w = jnp.concatenate([edge_weight, jnp.zeros((pad,), jnp.float32)])
  nblkt = EPAD // k
  return jnp.stack([src.reshape(nblkt, k), dst.reshape(nblkt, k),
                    w.view(jnp.int32).reshape(nblkt, k)], axis=1)


K0 = 128
K1 = 128


def kernel(x, edge_index_r0, edge_index_r1, edge_weight_r0, edge_weight_r1,
           Ws0_r0, Wn0_r0, b0_r0, Ws0_r1, Wn0_r1, b0_r1,
           Ws1_r0, Wn1_r0, b1_r0, Ws1_r1, Wn1_r1, b1_r1):
  ec0_r0 = _pad_edges(edge_index_r0, edge_weight_r0, K0)
  ec0_r1 = _pad_edges(edge_index_r1, edge_weight_r1, K0)
  ec1_r0 = _pad_edges(edge_index_r0, edge_weight_r0, K1)
  ec1_r1 = _pad_edges(edge_index_r1, edge_weight_r1, K1)
  z0 = jnp.zeros((ACC_ROWS, W0), jnp.float32)
  z1 = jnp.zeros((ACC_ROWS, W1), jnp.float32)

  w0all = jnp.concatenate([Wn0_r0, Wn0_r1, Ws0_r0 + Ws0_r1], axis=1)
  b0sum = (b0_r0 + b0_r1)[None, :]
  w1all = jnp.concatenate(
      [Wn1_r0, Wn1_r1, Ws1_r0 + Ws1_r1,
       jnp.zeros((HID, 128 - 3 * NC), jnp.float32)], axis=1)
  b1sum = (b1_r0 + b1_r1)[None, :]

  t0_r0, t0_r1, xs = _tc1(x, w0all, b0sum)
  acc0_r0, acc0_r1 = _make_sc_agg(W0, 8, K0)(t0_r0, t0_r1, ec0_r0, ec0_r1, z0)
  t1_r0, t1_r1, hs = _tc2(xs, acc0_r0, acc0_r1, w1all, b1sum)
  acc1_r0, acc1_r1 = _make_sc_agg(W1, 3, K1, True)(t1_r0, t1_r1, ec1_r0, ec1_r1, z1)
  return _tc3(hs, acc1_r0, acc1_r1)
